# baseline (speedup 1.0000x reference)
import functools

import jax
import jax.numpy as jnp
from jax import lax
from jax.experimental import pallas as pl
from jax.experimental.pallas import tpu as pltpu
from jax.experimental.pallas import tpu_sc as plsc

NC = 2
NS = 16
NW = NC * NS
CHUNK = 100


def _sc_mesh():
  return plsc.VectorSubcoreMesh(
      core_axis_name="c", subcore_axis_name="s", num_cores=NC, num_subcores=NS)


def _make_deg_kernel(E, NP):
  per_w = E // NW
  n_it = per_w // CHUNK
  rows_per_tile = NP // NS

  @functools.partial(
      pl.kernel,
      out_type=jax.ShapeDtypeStruct((NC, NP, 16), jnp.float32),
      mesh=_sc_mesh(),
      scratch_types=[
          pltpu.VMEM_SHARED((NP, 16), jnp.float32),
          pltpu.VMEM((n_it, CHUNK), jnp.int32),
          pltpu.VMEM((CHUNK, 16), jnp.float32),
          pltpu.VMEM((128, 16), jnp.float32),
      ],
  )
  def deg_kernel(col_hbm, out_hbm, acc, cbuf, ones_buf, zbuf):
    c = lax.axis_index("c")
    s = lax.axis_index("s")
    wid = c * NS + s
    zeros16 = jnp.zeros((16,), jnp.float32)
    ones16 = jnp.ones((16,), jnp.float32)

    def fill_body(i, _):
      zbuf[i] = zeros16
      return 0
    lax.fori_loop(0, 128, fill_body, 0)

    def ones_body(i, _):
      ones_buf[i] = ones16
      return 0
    lax.fori_loop(0, CHUNK, ones_body, 0)

    for k in range(rows_per_tile // 128):
      pltpu.sync_copy(zbuf, acc.at[pl.ds(s * rows_per_tile + k * 128, 128)])

    pltpu.sync_copy(col_hbm.at[wid], cbuf)
    plsc.subcore_barrier()

    def step(i, _):
      pltpu.sync_copy(ones_buf, acc.at[cbuf.at[i]], add=True)
      return 0
    lax.fori_loop(0, n_it, step, 0)

    plsc.subcore_barrier()
    pltpu.sync_copy(acc.at[pl.ds(s * rows_per_tile, rows_per_tile)],
                    out_hbm.at[c, pl.ds(s * rows_per_tile, rows_per_tile)])

  return deg_kernel


def _make_edge_kernel(E, NP, D):
  per_w = E // NW
  n_it = per_w // CHUNK
  rows_per_tile = NP // NS

  @functools.partial(
      pl.kernel,
      out_type=jax.ShapeDtypeStruct((NC, NP, D), jnp.float32),
      mesh=_sc_mesh(),
      scratch_types=[
          pltpu.VMEM_SHARED((NP, D), jnp.float32),
          pltpu.VMEM((1, CHUNK), jnp.int32),
          pltpu.VMEM((1, CHUNK), jnp.int32),
          pltpu.VMEM((1, CHUNK), jnp.int32),
          pltpu.VMEM((1, CHUNK), jnp.int32),
          pltpu.VMEM((CHUNK, D), jnp.float32),
          pltpu.VMEM((CHUNK, D), jnp.float32),
          pltpu.SemaphoreType.DMA,
          pltpu.SemaphoreType.DMA,
          pltpu.SemaphoreType.DMA,
          pltpu.SemaphoreType.DMA,
          pltpu.SemaphoreType.DMA,
          pltpu.SemaphoreType.DMA,
          pltpu.SemaphoreType.DMA,
          pltpu.SemaphoreType.DMA,
      ],
  )
  def edge_kernel(g_hbm, row_hbm, col_hbm, out_hbm, acc, rb0, rb1, cb0, cb1,
                  rows0, rows1, ir0, ir1, ic0, ic1, gs0, gs1, ss0, ss1):
    c = lax.axis_index("c")
    s = lax.axis_index("s")
    wid = c * NS + s
    zeros16 = jnp.zeros((16,), jnp.float32)

    def zero_body(i, _):
      for j in range(D // 16):
        rows0[i, pl.ds(j * 16, 16)] = zeros16
      return 0
    lax.fori_loop(0, CHUNK, zero_body, 0)
    nz, rem = rows_per_tile // CHUNK, rows_per_tile % CHUNK
    for k in range(nz):
      pltpu.sync_copy(rows0, acc.at[pl.ds(s * rows_per_tile + k * CHUNK, CHUNK)])
    if rem:
      pltpu.sync_copy(rows0.at[pl.ds(0, rem)],
                      acc.at[pl.ds(s * rows_per_tile + nz * CHUNK, rem)])
    plsc.subcore_barrier()

    def iload(src_hbm, j, buf, sem):
      pltpu.async_copy(src_hbm.at[wid, j], buf, sem)

    def iwait(buf, sem):
      pltpu.make_async_copy(row_hbm.at[0, 0], buf, sem).wait()

    def gather(ibuf, buf, sem):
      pltpu.async_copy(g_hbm.at[ibuf.at[0]], buf, sem)

    def gwait(ibuf, buf, sem):
      pltpu.make_async_copy(g_hbm.at[ibuf.at[0]], buf, sem).wait()

    def scat(ibuf, buf, sem):
      pltpu.async_copy(buf, acc.at[ibuf.at[0]], sem, add=True)

    def swait(ibuf, buf, sem):
      pltpu.make_async_copy(buf, acc.at[ibuf.at[0]], sem).wait()

    n_pair = n_it // 2
    iload(row_hbm, 0, rb0, ir0)
    iload(col_hbm, 0, cb0, ic0)
    iload(row_hbm, 1, rb1, ir1)
    iload(col_hbm, 1, cb1, ic1)
    iwait(rb0, ir0)
    gather(rb0, rows0, gs0)
    iwait(rb1, ir1)
    gather(rb1, rows1, gs1)

    def step(i, _):
      gwait(rb0, rows0, gs0)

      @pl.when(i < n_pair - 1)
      def _():
        iload(row_hbm, 2 * i + 2, rb0, ir0)
      iwait(cb0, ic0)
      scat(cb0, rows0, ss0)
      gwait(rb1, rows1, gs1)

      @pl.when(i < n_pair - 1)
      def _():
        iload(row_hbm, 2 * i + 3, rb1, ir1)
      iwait(cb1, ic1)
      scat(cb1, rows1, ss1)

      @pl.when(i < n_pair - 1)
      def _():
        swait(cb0, rows0, ss0)
        iload(col_hbm, 2 * i + 2, cb0, ic0)
        iwait(rb0, ir0)
        gather(rb0, rows0, gs0)
        swait(cb1, rows1, ss1)
        iload(col_hbm, 2 * i + 3, cb1, ic1)
        iwait(rb1, ir1)
        gather(rb1, rows1, gs1)
      return 0
    lax.fori_loop(0, n_pair, step, 0)

    swait(cb0, rows0, ss0)
    swait(cb1, rows1, ss1)

    plsc.subcore_barrier()
    pltpu.sync_copy(acc.at[pl.ds(s * rows_per_tile, rows_per_tile)],
                    out_hbm.at[c, pl.ds(s * rows_per_tile, rows_per_tile)])

  return edge_kernel


def _prescale_body(degp_ref, x_ref, dinv_ref, g_ref):
  dsum = degp_ref[0] + degp_ref[1]
  deg = jnp.sum(dsum, axis=1, keepdims=True) + 1.0
  dinv = lax.rsqrt(deg)
  dinv_ref[...] = dinv
  g_ref[...] = x_ref[...] * dinv


def _layer_body(parts_ref, g_ref, dinv_ref, w_ref, b_ref, gout_ref):
  g = g_ref[...]
  dinv = dinv_ref[...]
  agg = (parts_ref[0] + parts_ref[1] + g) * dinv
  h = jnp.maximum(
      lax.dot_general(agg, w_ref[...], (((1,), (0,)), ((), ())),
                      preferred_element_type=jnp.float32) + b_ref[...], 0.0)
  gout_ref[...] = h * dinv


def _final_body(N, BLK, parts_ref, g_ref, dinv_ref, w_ref, b_ref, wo_ref,
                bo_ref, out_ref, acc_ref):
  i = pl.program_id(0)

  @pl.when(i == 0)
  def _():
    acc_ref[...] = jnp.zeros_like(acc_ref)

  g = g_ref[...]
  dinv = dinv_ref[...]
  agg = (parts_ref[0] + parts_ref[1] + g) * dinv
  h = jnp.maximum(
      lax.dot_general(agg, w_ref[...], (((1,), (0,)), ((), ())),
                      preferred_element_type=jnp.float32) + b_ref[...], 0.0)
  rid = lax.broadcasted_iota(jnp.int32, h.shape, 0) + i * BLK
  h = jnp.where(rid < N, h, 0.0)
  acc_ref[...] += jnp.sum(h, axis=0, keepdims=True)
  pooled = acc_ref[...] * (1.0 / N)
  out_ref[...] = lax.dot_general(pooled, wo_ref[...], (((1,), (0,)), ((), ())),
                                 preferred_element_type=jnp.float32) + bo_ref[...]


def kernel(x, edge_index, W1, b1, W2, b2, W3, b3, Wout, bout):
  N, D = x.shape
  E = edge_index.shape[1]
  OUT = Wout.shape[1]
  NP = ((N + 1023) // 1024) * 1024
  BLK = 1024
  GRID = NP // BLK
  per_w = E // NW
  n_it = per_w // CHUNK

  row = edge_index[0].reshape(NW, n_it, 1, CHUNK)
  col = edge_index[1].reshape(NW, n_it, 1, CHUNK)
  x_pad = jnp.zeros((NP, D), x.dtype).at[:N].set(x)

  deg_parts = _make_deg_kernel(E, NP)(edge_index[1].reshape(NW, n_it, CHUNK))

  dinv, g = pl.pallas_call(
      _prescale_body,
      grid=(GRID,),
      in_specs=[
          pl.BlockSpec((NC, BLK, 16), lambda i: (0, i, 0)),
          pl.BlockSpec((BLK, D), lambda i: (i, 0)),
      ],
      out_specs=[
          pl.BlockSpec((BLK, 1), lambda i: (i, 0)),
          pl.BlockSpec((BLK, D), lambda i: (i, 0)),
      ],
      out_shape=[
          jax.ShapeDtypeStruct((NP, 1), jnp.float32),
          jax.ShapeDtypeStruct((NP, D), jnp.float32),
      ],
  )(deg_parts, x_pad)

  edge_kernel = _make_edge_kernel(E, NP, D)

  layer_call = pl.pallas_call(
      _layer_body,
      grid=(GRID,),
      in_specs=[
          pl.BlockSpec((NC, BLK, D), lambda i: (0, i, 0)),
          pl.BlockSpec((BLK, D), lambda i: (i, 0)),
          pl.BlockSpec((BLK, 1), lambda i: (i, 0)),
          pl.BlockSpec((D, D), lambda i: (0, 0)),
          pl.BlockSpec((1, D), lambda i: (0, 0)),
      ],
      out_specs=pl.BlockSpec((BLK, D), lambda i: (i, 0)),
      out_shape=jax.ShapeDtypeStruct((NP, D), jnp.float32),
  )

  for W, b in ((W1, b1), (W2, b2)):
    parts = edge_kernel(g, row, col)
    g = layer_call(parts, g, dinv, W, b.reshape(1, D))

  parts = edge_kernel(g, row, col)
  out = pl.pallas_call(
      functools.partial(_final_body, N, BLK),
      grid=(GRID,),
      in_specs=[
          pl.BlockSpec((NC, BLK, D), lambda i: (0, i, 0)),
          pl.BlockSpec((BLK, D), lambda i: (i, 0)),
          pl.BlockSpec((BLK, 1), lambda i: (i, 0)),
          pl.BlockSpec((D, D), lambda i: (0, 0)),
          pl.BlockSpec((1, D), lambda i: (0, 0)),
          pl.BlockSpec((D, OUT), lambda i: (0, 0)),
          pl.BlockSpec((1, OUT), lambda i: (0, 0)),
      ],
      out_specs=pl.BlockSpec((1, OUT), lambda i: (0, 0)),
      out_shape=jax.ShapeDtypeStruct((1, OUT), jnp.float32),
      scratch_shapes=[pltpu.VMEM((1, D), jnp.float32)],
  )(parts, g, dinv, W3, b3.reshape(1, D), Wout, bout.reshape(1, OUT))
  return out


# speedup vs baseline: 1.2382x; 1.2382x over previous
import functools

import jax
import jax.numpy as jnp
from jax import lax
from jax.experimental import pallas as pl
from jax.experimental.pallas import tpu as pltpu
from jax.experimental.pallas import tpu_sc as plsc

NC = 2
NS = 16
NW = NC * NS
CHUNK = 100


def _sc_mesh():
  return plsc.VectorSubcoreMesh(
      core_axis_name="c", subcore_axis_name="s", num_cores=NC, num_subcores=NS)


def _make_deg_kernel(E, NP):
  per_w = E // NW
  n_it = per_w // CHUNK
  rows_per_tile = NP // NS

  @functools.partial(
      pl.kernel,
      out_type=jax.ShapeDtypeStruct((NC, NP, 16), jnp.float32),
      mesh=_sc_mesh(),
      scratch_types=[
          pltpu.VMEM_SHARED((NP, 16), jnp.float32),
          pltpu.VMEM((n_it, CHUNK), jnp.int32),
          pltpu.VMEM((CHUNK, 16), jnp.float32),
          pltpu.VMEM((128, 16), jnp.float32),
      ],
  )
  def deg_kernel(col_hbm, out_hbm, acc, cbuf, ones_buf, zbuf):
    c = lax.axis_index("c")
    s = lax.axis_index("s")
    wid = c * NS + s
    zeros16 = jnp.zeros((16,), jnp.float32)
    ones16 = jnp.ones((16,), jnp.float32)

    def fill_body(i, _):
      zbuf[i] = zeros16
      return 0
    lax.fori_loop(0, 128, fill_body, 0)

    def ones_body(i, _):
      ones_buf[i] = ones16
      return 0
    lax.fori_loop(0, CHUNK, ones_body, 0)

    for k in range(rows_per_tile // 128):
      pltpu.sync_copy(zbuf, acc.at[pl.ds(s * rows_per_tile + k * 128, 128)])

    pltpu.sync_copy(col_hbm.at[wid], cbuf)
    plsc.subcore_barrier()

    def step(i, _):
      pltpu.sync_copy(ones_buf, acc.at[cbuf.at[i]], add=True)
      return 0
    lax.fori_loop(0, n_it, step, 0)

    plsc.subcore_barrier()
    pltpu.sync_copy(acc.at[pl.ds(s * rows_per_tile, rows_per_tile)],
                    out_hbm.at[c, pl.ds(s * rows_per_tile, rows_per_tile)])

  return deg_kernel


def _make_edge_kernel(E, NP, D):
  per_w = E // NW
  n_it = per_w // CHUNK
  rows_per_tile = NP // NS

  @functools.partial(
      pl.kernel,
      out_type=jax.ShapeDtypeStruct((NC, NP, D), jnp.float32),
      mesh=_sc_mesh(),
      scratch_types=[
          pltpu.VMEM_SHARED((NP, D), jnp.float32),
          pltpu.VMEM((1, CHUNK), jnp.int32),
          pltpu.VMEM((1, CHUNK), jnp.int32),
          pltpu.VMEM((1, CHUNK), jnp.int32),
          pltpu.VMEM((1, CHUNK), jnp.int32),
          pltpu.VMEM((CHUNK, D), jnp.float32),
          pltpu.VMEM((CHUNK, D), jnp.float32),
          pltpu.SemaphoreType.DMA,
          pltpu.SemaphoreType.DMA,
          pltpu.SemaphoreType.DMA,
          pltpu.SemaphoreType.DMA,
          pltpu.SemaphoreType.DMA,
          pltpu.SemaphoreType.DMA,
          pltpu.SemaphoreType.DMA,
          pltpu.SemaphoreType.DMA,
      ],
  )
  def edge_kernel(g_hbm, row_hbm, col_hbm, out_hbm, acc, rb0, rb1, cb0, cb1,
                  rows0, rows1, ir0, ir1, ic0, ic1, gs0, gs1, ss0, ss1):
    c = lax.axis_index("c")
    s = lax.axis_index("s")
    wid = c * NS + s
    zeros16 = jnp.zeros((16,), jnp.float32)

    def zero_body(i, _):
      for j in range(D // 16):
        rows0[i, pl.ds(j * 16, 16)] = zeros16
      return 0
    lax.fori_loop(0, CHUNK, zero_body, 0)
    nz, rem = rows_per_tile // CHUNK, rows_per_tile % CHUNK
    for k in range(nz):
      pltpu.sync_copy(rows0, acc.at[pl.ds(s * rows_per_tile + k * CHUNK, CHUNK)])
    if rem:
      pltpu.sync_copy(rows0.at[pl.ds(0, rem)],
                      acc.at[pl.ds(s * rows_per_tile + nz * CHUNK, rem)])
    plsc.subcore_barrier()

    def iload(src_hbm, j, buf, sem):
      pltpu.async_copy(src_hbm.at[wid, j], buf, sem)

    def iwait(buf, sem):
      pltpu.make_async_copy(row_hbm.at[0, 0], buf, sem).wait()

    def gather(ibuf, buf, sem):
      pltpu.async_copy(g_hbm.at[ibuf.at[0]], buf, sem)

    def gwait(ibuf, buf, sem):
      pltpu.make_async_copy(g_hbm.at[ibuf.at[0]], buf, sem).wait()

    def scat(ibuf, buf, sem):
      pltpu.async_copy(buf, acc.at[ibuf.at[0]], sem, add=True)

    def swait(ibuf, buf, sem):
      pltpu.make_async_copy(buf, acc.at[ibuf.at[0]], sem).wait()

    n_pair = n_it // 2
    iload(row_hbm, 0, rb0, ir0)
    iload(col_hbm, 0, cb0, ic0)
    iload(row_hbm, 1, rb1, ir1)
    iload(col_hbm, 1, cb1, ic1)
    iwait(rb0, ir0)
    gather(rb0, rows0, gs0)
    iwait(rb1, ir1)
    gather(rb1, rows1, gs1)

    def step(i, _):
      gwait(rb0, rows0, gs0)

      @pl.when(i < n_pair - 1)
      def _():
        iload(row_hbm, 2 * i + 2, rb0, ir0)
      iwait(cb0, ic0)
      scat(cb0, rows0, ss0)
      swait(cb0, rows0, ss0)

      @pl.when(i < n_pair - 1)
      def _():
        iload(col_hbm, 2 * i + 2, cb0, ic0)
        iwait(rb0, ir0)
        gather(rb0, rows0, gs0)

      gwait(rb1, rows1, gs1)

      @pl.when(i < n_pair - 1)
      def _():
        iload(row_hbm, 2 * i + 3, rb1, ir1)
      iwait(cb1, ic1)
      scat(cb1, rows1, ss1)
      swait(cb1, rows1, ss1)

      @pl.when(i < n_pair - 1)
      def _():
        iload(col_hbm, 2 * i + 3, cb1, ic1)
        iwait(rb1, ir1)
        gather(rb1, rows1, gs1)
      return 0
    lax.fori_loop(0, n_pair, step, 0)

    plsc.subcore_barrier()
    pltpu.sync_copy(acc.at[pl.ds(s * rows_per_tile, rows_per_tile)],
                    out_hbm.at[c, pl.ds(s * rows_per_tile, rows_per_tile)])

  return edge_kernel


def _prescale_body(degp_ref, x_ref, dinv_ref, g_ref):
  dsum = degp_ref[0] + degp_ref[1]
  deg = jnp.sum(dsum, axis=1, keepdims=True) + 1.0
  dinv = lax.rsqrt(deg)
  dinv_ref[...] = dinv
  g_ref[...] = x_ref[...] * dinv


def _layer_body(parts_ref, g_ref, dinv_ref, w_ref, b_ref, gout_ref):
  g = g_ref[...]
  dinv = dinv_ref[...]
  agg = (parts_ref[0] + parts_ref[1] + g) * dinv
  h = jnp.maximum(
      lax.dot_general(agg, w_ref[...], (((1,), (0,)), ((), ())),
                      preferred_element_type=jnp.float32) + b_ref[...], 0.0)
  gout_ref[...] = h * dinv


def _final_body(N, BLK, parts_ref, g_ref, dinv_ref, w_ref, b_ref, wo_ref,
                bo_ref, out_ref, acc_ref):
  i = pl.program_id(0)

  @pl.when(i == 0)
  def _():
    acc_ref[...] = jnp.zeros_like(acc_ref)

  g = g_ref[...]
  dinv = dinv_ref[...]
  agg = (parts_ref[0] + parts_ref[1] + g) * dinv
  h = jnp.maximum(
      lax.dot_general(agg, w_ref[...], (((1,), (0,)), ((), ())),
                      preferred_element_type=jnp.float32) + b_ref[...], 0.0)
  rid = lax.broadcasted_iota(jnp.int32, h.shape, 0) + i * BLK
  h = jnp.where(rid < N, h, 0.0)
  acc_ref[...] += jnp.sum(h, axis=0, keepdims=True)
  pooled = acc_ref[...] * (1.0 / N)
  out_ref[...] = lax.dot_general(pooled, wo_ref[...], (((1,), (0,)), ((), ())),
                                 preferred_element_type=jnp.float32) + bo_ref[...]


def kernel(x, edge_index, W1, b1, W2, b2, W3, b3, Wout, bout):
  N, D = x.shape
  E = edge_index.shape[1]
  OUT = Wout.shape[1]
  NP = ((N + 1023) // 1024) * 1024
  BLK = 1024
  GRID = NP // BLK
  per_w = E // NW
  n_it = per_w // CHUNK

  row = edge_index[0].reshape(NW, n_it, 1, CHUNK)
  col = edge_index[1].reshape(NW, n_it, 1, CHUNK)
  x_pad = jnp.zeros((NP, D), x.dtype).at[:N].set(x)

  deg_parts = _make_deg_kernel(E, NP)(edge_index[1].reshape(NW, n_it, CHUNK))

  dinv, g = pl.pallas_call(
      _prescale_body,
      grid=(GRID,),
      in_specs=[
          pl.BlockSpec((NC, BLK, 16), lambda i: (0, i, 0)),
          pl.BlockSpec((BLK, D), lambda i: (i, 0)),
      ],
      out_specs=[
          pl.BlockSpec((BLK, 1), lambda i: (i, 0)),
          pl.BlockSpec((BLK, D), lambda i: (i, 0)),
      ],
      out_shape=[
          jax.ShapeDtypeStruct((NP, 1), jnp.float32),
          jax.ShapeDtypeStruct((NP, D), jnp.float32),
      ],
  )(deg_parts, x_pad)

  edge_kernel = _make_edge_kernel(E, NP, D)

  layer_call = pl.pallas_call(
      _layer_body,
      grid=(GRID,),
      in_specs=[
          pl.BlockSpec((NC, BLK, D), lambda i: (0, i, 0)),
          pl.BlockSpec((BLK, D), lambda i: (i, 0)),
          pl.BlockSpec((BLK, 1), lambda i: (i, 0)),
          pl.BlockSpec((D, D), lambda i: (0, 0)),
          pl.BlockSpec((1, D), lambda i: (0, 0)),
      ],
      out_specs=pl.BlockSpec((BLK, D), lambda i: (i, 0)),
      out_shape=jax.ShapeDtypeStruct((NP, D), jnp.float32),
  )

  for W, b in ((W1, b1), (W2, b2)):
    parts = edge_kernel(g, row, col)
    g = layer_call(parts, g, dinv, W, b.reshape(1, D))

  parts = edge_kernel(g, row, col)
  out = pl.pallas_call(
      functools.partial(_final_body, N, BLK),
      grid=(GRID,),
      in_specs=[
          pl.BlockSpec((NC, BLK, D), lambda i: (0, i, 0)),
          pl.BlockSpec((BLK, D), lambda i: (i, 0)),
          pl.BlockSpec((BLK, 1), lambda i: (i, 0)),
          pl.BlockSpec((D, D), lambda i: (0, 0)),
          pl.BlockSpec((1, D), lambda i: (0, 0)),
          pl.BlockSpec((D, OUT), lambda i: (0, 0)),
          pl.BlockSpec((1, OUT), lambda i: (0, 0)),
      ],
      out_specs=pl.BlockSpec((1, OUT), lambda i: (0, 0)),
      out_shape=jax.ShapeDtypeStruct((1, OUT), jnp.float32),
      scratch_shapes=[pltpu.VMEM((1, D), jnp.float32)],
  )(parts, g, dinv, W3, b3.reshape(1, D), Wout, bout.reshape(1, OUT))
  return out


# speedup vs baseline: 1.2956x; 1.0464x over previous
import functools

import jax
import jax.numpy as jnp
from jax import lax
from jax.experimental import pallas as pl
from jax.experimental.pallas import tpu as pltpu
from jax.experimental.pallas import tpu_sc as plsc

NC = 2
NS = 16
NW = NC * NS
CHUNK = 125


def _sc_mesh():
  return plsc.VectorSubcoreMesh(
      core_axis_name="c", subcore_axis_name="s", num_cores=NC, num_subcores=NS)


def _make_deg_kernel(E, NP):
  per_w = E // NW
  n_it = per_w // CHUNK
  rows_per_tile = NP // NS

  @functools.partial(
      pl.kernel,
      out_type=jax.ShapeDtypeStruct((NC, NP, 16), jnp.float32),
      mesh=_sc_mesh(),
      scratch_types=[
          pltpu.VMEM_SHARED((NP, 16), jnp.float32),
          pltpu.VMEM((n_it, CHUNK), jnp.int32),
          pltpu.VMEM((CHUNK, 16), jnp.float32),
          pltpu.VMEM((128, 16), jnp.float32),
      ],
  )
  def deg_kernel(col_hbm, out_hbm, acc, cbuf, ones_buf, zbuf):
    c = lax.axis_index("c")
    s = lax.axis_index("s")
    wid = c * NS + s
    zeros16 = jnp.zeros((16,), jnp.float32)
    ones16 = jnp.ones((16,), jnp.float32)

    def fill_body(i, _):
      zbuf[i] = zeros16
      return 0
    lax.fori_loop(0, 128, fill_body, 0)

    def ones_body(i, _):
      ones_buf[i] = ones16
      return 0
    lax.fori_loop(0, CHUNK, ones_body, 0)

    for k in range(rows_per_tile // 128):
      pltpu.sync_copy(zbuf, acc.at[pl.ds(s * rows_per_tile + k * 128, 128)])

    pltpu.sync_copy(col_hbm.at[wid], cbuf)
    plsc.subcore_barrier()

    def step(i, _):
      pltpu.sync_copy(ones_buf, acc.at[cbuf.at[i]], add=True)
      return 0
    lax.fori_loop(0, n_it, step, 0)

    plsc.subcore_barrier()
    pltpu.sync_copy(acc.at[pl.ds(s * rows_per_tile, rows_per_tile)],
                    out_hbm.at[c, pl.ds(s * rows_per_tile, rows_per_tile)])

  return deg_kernel


def _make_edge_kernel(E, NP, D):
  per_w = E // NW
  n_it = per_w // CHUNK
  rows_per_tile = NP // NS

  @functools.partial(
      pl.kernel,
      out_type=jax.ShapeDtypeStruct((NC, NP, D), jnp.float32),
      mesh=_sc_mesh(),
      scratch_types=[
          pltpu.VMEM_SHARED((NP, D), jnp.float32),
          pltpu.VMEM((1, CHUNK), jnp.int32),
          pltpu.VMEM((1, CHUNK), jnp.int32),
          pltpu.VMEM((1, CHUNK), jnp.int32),
          pltpu.VMEM((1, CHUNK), jnp.int32),
          pltpu.VMEM((CHUNK, D), jnp.float32),
          pltpu.VMEM((CHUNK, D), jnp.float32),
          pltpu.SemaphoreType.DMA,
          pltpu.SemaphoreType.DMA,
          pltpu.SemaphoreType.DMA,
          pltpu.SemaphoreType.DMA,
          pltpu.SemaphoreType.DMA,
          pltpu.SemaphoreType.DMA,
          pltpu.SemaphoreType.DMA,
          pltpu.SemaphoreType.DMA,
      ],
  )
  def edge_kernel(g_hbm, row_hbm, col_hbm, out_hbm, acc, rb0, rb1, cb0, cb1,
                  rows0, rows1, ir0, ir1, ic0, ic1, gs0, gs1, ss0, ss1):
    c = lax.axis_index("c")
    s = lax.axis_index("s")
    wid = c * NS + s
    zeros16 = jnp.zeros((16,), jnp.float32)

    def zero_body(i, _):
      for j in range(D // 16):
        rows0[i, pl.ds(j * 16, 16)] = zeros16
      return 0
    lax.fori_loop(0, CHUNK, zero_body, 0)
    nz, rem = rows_per_tile // CHUNK, rows_per_tile % CHUNK
    for k in range(nz):
      pltpu.sync_copy(rows0, acc.at[pl.ds(s * rows_per_tile + k * CHUNK, CHUNK)])
    if rem:
      pltpu.sync_copy(rows0.at[pl.ds(0, rem)],
                      acc.at[pl.ds(s * rows_per_tile + nz * CHUNK, rem)])
    plsc.subcore_barrier()

    def iload(src_hbm, j, buf, sem):
      pltpu.async_copy(src_hbm.at[wid, j], buf, sem)

    def iwait(buf, sem):
      pltpu.make_async_copy(row_hbm.at[0, 0], buf, sem).wait()

    def gather(ibuf, buf, sem):
      pltpu.async_copy(g_hbm.at[ibuf.at[0]], buf, sem)

    def gwait(ibuf, buf, sem):
      pltpu.make_async_copy(g_hbm.at[ibuf.at[0]], buf, sem).wait()

    def scat(ibuf, buf, sem):
      pltpu.async_copy(buf, acc.at[ibuf.at[0]], sem, add=True)

    def swait(ibuf, buf, sem):
      pltpu.make_async_copy(buf, acc.at[ibuf.at[0]], sem).wait()

    n_pair = n_it // 2
    iload(row_hbm, 0, rb0, ir0)
    iload(col_hbm, 0, cb0, ic0)
    iload(row_hbm, 1, rb1, ir1)
    iload(col_hbm, 1, cb1, ic1)
    iwait(rb0, ir0)
    gather(rb0, rows0, gs0)
    iwait(rb1, ir1)
    gather(rb1, rows1, gs1)

    def step(i, _):
      gwait(rb0, rows0, gs0)

      @pl.when(i < n_pair - 1)
      def _():
        iload(row_hbm, 2 * i + 2, rb0, ir0)
      iwait(cb0, ic0)
      scat(cb0, rows0, ss0)
      swait(cb0, rows0, ss0)

      @pl.when(i < n_pair - 1)
      def _():
        iload(col_hbm, 2 * i + 2, cb0, ic0)
        iwait(rb0, ir0)
        gather(rb0, rows0, gs0)

      gwait(rb1, rows1, gs1)

      @pl.when(i < n_pair - 1)
      def _():
        iload(row_hbm, 2 * i + 3, rb1, ir1)
      iwait(cb1, ic1)
      scat(cb1, rows1, ss1)
      swait(cb1, rows1, ss1)

      @pl.when(i < n_pair - 1)
      def _():
        iload(col_hbm, 2 * i + 3, cb1, ic1)
        iwait(rb1, ir1)
        gather(rb1, rows1, gs1)
      return 0
    lax.fori_loop(0, n_pair, step, 0)

    plsc.subcore_barrier()
    pltpu.sync_copy(acc.at[pl.ds(s * rows_per_tile, rows_per_tile)],
                    out_hbm.at[c, pl.ds(s * rows_per_tile, rows_per_tile)])

  return edge_kernel


def _prescale_body(degp_ref, x_ref, dinv_ref, g_ref):
  dsum = degp_ref[0] + degp_ref[1]
  deg = jnp.sum(dsum, axis=1, keepdims=True) + 1.0
  dinv = lax.rsqrt(deg)
  dinv_ref[...] = dinv
  g_ref[...] = x_ref[...] * dinv


def _layer_body(parts_ref, g_ref, dinv_ref, w_ref, b_ref, gout_ref):
  g = g_ref[...]
  dinv = dinv_ref[...]
  agg = (parts_ref[0] + parts_ref[1] + g) * dinv
  h = jnp.maximum(
      lax.dot_general(agg, w_ref[...], (((1,), (0,)), ((), ())),
                      preferred_element_type=jnp.float32) + b_ref[...], 0.0)
  gout_ref[...] = h * dinv


def _final_body(N, BLK, parts_ref, g_ref, dinv_ref, w_ref, b_ref, wo_ref,
                bo_ref, out_ref, acc_ref):
  i = pl.program_id(0)

  @pl.when(i == 0)
  def _():
    acc_ref[...] = jnp.zeros_like(acc_ref)

  g = g_ref[...]
  dinv = dinv_ref[...]
  agg = (parts_ref[0] + parts_ref[1] + g) * dinv
  h = jnp.maximum(
      lax.dot_general(agg, w_ref[...], (((1,), (0,)), ((), ())),
                      preferred_element_type=jnp.float32) + b_ref[...], 0.0)
  rid = lax.broadcasted_iota(jnp.int32, h.shape, 0) + i * BLK
  h = jnp.where(rid < N, h, 0.0)
  acc_ref[...] += jnp.sum(h, axis=0, keepdims=True)
  pooled = acc_ref[...] * (1.0 / N)
  out_ref[...] = lax.dot_general(pooled, wo_ref[...], (((1,), (0,)), ((), ())),
                                 preferred_element_type=jnp.float32) + bo_ref[...]


def kernel(x, edge_index, W1, b1, W2, b2, W3, b3, Wout, bout):
  N, D = x.shape
  E = edge_index.shape[1]
  OUT = Wout.shape[1]
  NP = ((N + 1023) // 1024) * 1024
  BLK = 1024
  GRID = NP // BLK
  per_w = E // NW
  n_it = per_w // CHUNK

  row = edge_index[0].reshape(NW, n_it, 1, CHUNK)
  col = edge_index[1].reshape(NW, n_it, 1, CHUNK)
  x_pad = jnp.zeros((NP, D), x.dtype).at[:N].set(x)

  deg_parts = _make_deg_kernel(E, NP)(edge_index[1].reshape(NW, -1, CHUNK))

  dinv, g = pl.pallas_call(
      _prescale_body,
      grid=(GRID,),
      in_specs=[
          pl.BlockSpec((NC, BLK, 16), lambda i: (0, i, 0)),
          pl.BlockSpec((BLK, D), lambda i: (i, 0)),
      ],
      out_specs=[
          pl.BlockSpec((BLK, 1), lambda i: (i, 0)),
          pl.BlockSpec((BLK, D), lambda i: (i, 0)),
      ],
      out_shape=[
          jax.ShapeDtypeStruct((NP, 1), jnp.float32),
          jax.ShapeDtypeStruct((NP, D), jnp.float32),
      ],
  )(deg_parts, x_pad)

  edge_kernel = _make_edge_kernel(E, NP, D)

  layer_call = pl.pallas_call(
      _layer_body,
      grid=(GRID,),
      in_specs=[
          pl.BlockSpec((NC, BLK, D), lambda i: (0, i, 0)),
          pl.BlockSpec((BLK, D), lambda i: (i, 0)),
          pl.BlockSpec((BLK, 1), lambda i: (i, 0)),
          pl.BlockSpec((D, D), lambda i: (0, 0)),
          pl.BlockSpec((1, D), lambda i: (0, 0)),
      ],
      out_specs=pl.BlockSpec((BLK, D), lambda i: (i, 0)),
      out_shape=jax.ShapeDtypeStruct((NP, D), jnp.float32),
  )

  for W, b in ((W1, b1), (W2, b2)):
    parts = edge_kernel(g, row, col)
    g = layer_call(parts, g, dinv, W, b.reshape(1, D))

  parts = edge_kernel(g, row, col)
  out = pl.pallas_call(
      functools.partial(_final_body, N, BLK),
      grid=(GRID,),
      in_specs=[
          pl.BlockSpec((NC, BLK, D), lambda i: (0, i, 0)),
          pl.BlockSpec((BLK, D), lambda i: (i, 0)),
          pl.BlockSpec((BLK, 1), lambda i: (i, 0)),
          pl.BlockSpec((D, D), lambda i: (0, 0)),
          pl.BlockSpec((1, D), lambda i: (0, 0)),
          pl.BlockSpec((D, OUT), lambda i: (0, 0)),
          pl.BlockSpec((1, OUT), lambda i: (0, 0)),
      ],
      out_specs=pl.BlockSpec((1, OUT), lambda i: (0, 0)),
      out_shape=jax.ShapeDtypeStruct((1, OUT), jnp.float32),
      scratch_shapes=[pltpu.VMEM((1, D), jnp.float32)],
  )(parts, g, dinv, W3, b3.reshape(1, D), Wout, bout.reshape(1, OUT))
  return out


# speedup vs baseline: 1.3054x; 1.0075x over previous
import functools

import jax
import jax.numpy as jnp
from jax import lax
from jax.experimental import pallas as pl
from jax.experimental.pallas import tpu as pltpu
from jax.experimental.pallas import tpu_sc as plsc

NC = 2
NS = 16
NW = NC * NS
CHUNK = 125


def _sc_mesh():
  return plsc.VectorSubcoreMesh(
      core_axis_name="c", subcore_axis_name="s", num_cores=NC, num_subcores=NS)


def _make_deg_kernel(E, NP):
  per_w = E // NW
  n_it = per_w // CHUNK
  rows_per_tile = NP // NS

  @functools.partial(
      pl.kernel,
      out_type=jax.ShapeDtypeStruct((NC, NP, 16), jnp.float32),
      mesh=_sc_mesh(),
      scratch_types=[
          pltpu.VMEM_SHARED((NP, 16), jnp.float32),
          pltpu.VMEM((n_it, CHUNK), jnp.int32),
          pltpu.VMEM((CHUNK, 16), jnp.float32),
          pltpu.VMEM((128, 16), jnp.float32),
      ],
  )
  def deg_kernel(col_hbm, out_hbm, acc, cbuf, ones_buf, zbuf):
    c = lax.axis_index("c")
    s = lax.axis_index("s")
    wid = c * NS + s
    zeros16 = jnp.zeros((16,), jnp.float32)
    ones16 = jnp.ones((16,), jnp.float32)

    def fill_body(i, _):
      zbuf[i] = zeros16
      return 0
    lax.fori_loop(0, 128, fill_body, 0)

    def ones_body(i, _):
      ones_buf[i] = ones16
      return 0
    lax.fori_loop(0, CHUNK, ones_body, 0)

    for k in range(rows_per_tile // 128):
      pltpu.sync_copy(zbuf, acc.at[pl.ds(s * rows_per_tile + k * 128, 128)])

    pltpu.sync_copy(col_hbm.at[wid], cbuf)
    plsc.subcore_barrier()

    def step(i, _):
      pltpu.sync_copy(ones_buf, acc.at[cbuf.at[i]], add=True)
      return 0
    lax.fori_loop(0, n_it, step, 0)

    plsc.subcore_barrier()
    pltpu.sync_copy(acc.at[pl.ds(s * rows_per_tile, rows_per_tile)],
                    out_hbm.at[c, pl.ds(s * rows_per_tile, rows_per_tile)])

  return deg_kernel


def _make_edge_kernel(E, NP, D):
  per_w = E // NW
  n_it = per_w // CHUNK
  rows_per_tile = NP // NS

  @functools.partial(
      pl.kernel,
      out_type=jax.ShapeDtypeStruct((NC, NP, D), jnp.float32),
      mesh=_sc_mesh(),
      scratch_types=[
          pltpu.VMEM_SHARED((NP, D), jnp.float32),
          pltpu.VMEM((1, CHUNK), jnp.int32),
          pltpu.VMEM((1, CHUNK), jnp.int32),
          pltpu.VMEM((1, CHUNK), jnp.int32),
          pltpu.VMEM((1, CHUNK), jnp.int32),
          pltpu.VMEM((CHUNK, D), jnp.float32),
          pltpu.VMEM((CHUNK, D), jnp.float32),
          pltpu.SemaphoreType.DMA,
          pltpu.SemaphoreType.DMA,
          pltpu.SemaphoreType.DMA,
          pltpu.SemaphoreType.DMA,
          pltpu.SemaphoreType.DMA,
          pltpu.SemaphoreType.DMA,
          pltpu.SemaphoreType.DMA,
          pltpu.SemaphoreType.DMA,
      ],
  )
  def edge_kernel(g_hbm, row_hbm, col_hbm, out_hbm, acc, rb0, rb1, cb0, cb1,
                  rows0, rows1, ir0, ir1, ic0, ic1, gs0, gs1, ss0, ss1):
    c = lax.axis_index("c")
    s = lax.axis_index("s")
    wid = c * NS + s
    zeros16 = jnp.zeros((16,), jnp.float32)


    def iload(src_hbm, j, buf, sem):
      pltpu.async_copy(src_hbm.at[wid, j], buf, sem)

    def iwait(buf, sem):
      pltpu.make_async_copy(row_hbm.at[0, 0], buf, sem).wait()

    def gather(ibuf, buf, sem):
      pltpu.async_copy(g_hbm.at[ibuf.at[0]], buf, sem)

    def gwait(ibuf, buf, sem):
      pltpu.make_async_copy(g_hbm.at[ibuf.at[0]], buf, sem).wait()

    def scat(ibuf, buf, sem):
      pltpu.async_copy(buf, acc.at[ibuf.at[0]], sem, add=True)

    def swait(ibuf, buf, sem):
      pltpu.make_async_copy(buf, acc.at[ibuf.at[0]], sem).wait()

    n_pair = n_it // 2
    iload(row_hbm, 0, rb0, ir0)
    iload(col_hbm, 0, cb0, ic0)
    iload(row_hbm, 1, rb1, ir1)
    iload(col_hbm, 1, cb1, ic1)

    def zero_body(i, _):
      for j in range(D // 16):
        rows0[i, pl.ds(j * 16, 16)] = zeros16
      return 0
    lax.fori_loop(0, CHUNK, zero_body, 0)
    iwait(rb1, ir1)
    gather(rb1, rows1, gs1)
    nz, rem = rows_per_tile // CHUNK, rows_per_tile % CHUNK
    for k in range(nz):
      pltpu.sync_copy(rows0, acc.at[pl.ds(s * rows_per_tile + k * CHUNK, CHUNK)])
    if rem:
      pltpu.sync_copy(rows0.at[pl.ds(0, rem)],
                      acc.at[pl.ds(s * rows_per_tile + nz * CHUNK, rem)])
    iwait(rb0, ir0)
    gather(rb0, rows0, gs0)
    plsc.subcore_barrier()

    def step(i, _):
      gwait(rb0, rows0, gs0)

      @pl.when(i < n_pair - 1)
      def _():
        iload(row_hbm, 2 * i + 2, rb0, ir0)
      iwait(cb0, ic0)
      scat(cb0, rows0, ss0)
      swait(cb0, rows0, ss0)

      @pl.when(i < n_pair - 1)
      def _():
        iload(col_hbm, 2 * i + 2, cb0, ic0)
        iwait(rb0, ir0)
        gather(rb0, rows0, gs0)

      gwait(rb1, rows1, gs1)

      @pl.when(i < n_pair - 1)
      def _():
        iload(row_hbm, 2 * i + 3, rb1, ir1)
      iwait(cb1, ic1)
      scat(cb1, rows1, ss1)
      swait(cb1, rows1, ss1)

      @pl.when(i < n_pair - 1)
      def _():
        iload(col_hbm, 2 * i + 3, cb1, ic1)
        iwait(rb1, ir1)
        gather(rb1, rows1, gs1)
      return 0
    lax.fori_loop(0, n_pair, step, 0)

    plsc.subcore_barrier()
    pltpu.sync_copy(acc.at[pl.ds(s * rows_per_tile, rows_per_tile)],
                    out_hbm.at[c, pl.ds(s * rows_per_tile, rows_per_tile)])

  return edge_kernel


def _prescale_body(degp_ref, x_ref, dinv_ref, g_ref):
  dsum = degp_ref[0] + degp_ref[1]
  deg = jnp.sum(dsum, axis=1, keepdims=True) + 1.0
  dinv = lax.rsqrt(deg)
  dinv_ref[...] = dinv
  g_ref[...] = x_ref[...] * dinv


def _layer_body(parts_ref, g_ref, dinv_ref, w_ref, b_ref, gout_ref):
  g = g_ref[...]
  dinv = dinv_ref[...]
  agg = (parts_ref[0] + parts_ref[1] + g) * dinv
  h = jnp.maximum(
      lax.dot_general(agg, w_ref[...], (((1,), (0,)), ((), ())),
                      preferred_element_type=jnp.float32) + b_ref[...], 0.0)
  gout_ref[...] = h * dinv


def _final_body(N, BLK, parts_ref, g_ref, dinv_ref, w_ref, b_ref, wo_ref,
                bo_ref, out_ref, acc_ref):
  i = pl.program_id(0)

  @pl.when(i == 0)
  def _():
    acc_ref[...] = jnp.zeros_like(acc_ref)

  g = g_ref[...]
  dinv = dinv_ref[...]
  agg = (parts_ref[0] + parts_ref[1] + g) * dinv
  h = jnp.maximum(
      lax.dot_general(agg, w_ref[...], (((1,), (0,)), ((), ())),
                      preferred_element_type=jnp.float32) + b_ref[...], 0.0)
  rid = lax.broadcasted_iota(jnp.int32, h.shape, 0) + i * BLK
  h = jnp.where(rid < N, h, 0.0)
  acc_ref[...] += jnp.sum(h, axis=0, keepdims=True)
  pooled = acc_ref[...] * (1.0 / N)
  out_ref[...] = lax.dot_general(pooled, wo_ref[...], (((1,), (0,)), ((), ())),
                                 preferred_element_type=jnp.float32) + bo_ref[...]


def kernel(x, edge_index, W1, b1, W2, b2, W3, b3, Wout, bout):
  N, D = x.shape
  E = edge_index.shape[1]
  OUT = Wout.shape[1]
  NP = ((N + 1023) // 1024) * 1024
  BLK = 1024
  GRID = NP // BLK
  per_w = E // NW
  n_it = per_w // CHUNK

  row = edge_index[0].reshape(NW, n_it, 1, CHUNK)
  col = edge_index[1].reshape(NW, n_it, 1, CHUNK)
  x_pad = jnp.zeros((NP, D), x.dtype).at[:N].set(x)

  deg_parts = _make_deg_kernel(E, NP)(edge_index[1].reshape(NW, -1, CHUNK))

  dinv, g = pl.pallas_call(
      _prescale_body,
      grid=(GRID,),
      in_specs=[
          pl.BlockSpec((NC, BLK, 16), lambda i: (0, i, 0)),
          pl.BlockSpec((BLK, D), lambda i: (i, 0)),
      ],
      out_specs=[
          pl.BlockSpec((BLK, 1), lambda i: (i, 0)),
          pl.BlockSpec((BLK, D), lambda i: (i, 0)),
      ],
      out_shape=[
          jax.ShapeDtypeStruct((NP, 1), jnp.float32),
          jax.ShapeDtypeStruct((NP, D), jnp.float32),
      ],
  )(deg_parts, x_pad)

  edge_kernel = _make_edge_kernel(E, NP, D)

  layer_call = pl.pallas_call(
      _layer_body,
      grid=(GRID,),
      in_specs=[
          pl.BlockSpec((NC, BLK, D), lambda i: (0, i, 0)),
          pl.BlockSpec((BLK, D), lambda i: (i, 0)),
          pl.BlockSpec((BLK, 1), lambda i: (i, 0)),
          pl.BlockSpec((D, D), lambda i: (0, 0)),
          pl.BlockSpec((1, D), lambda i: (0, 0)),
      ],
      out_specs=pl.BlockSpec((BLK, D), lambda i: (i, 0)),
      out_shape=jax.ShapeDtypeStruct((NP, D), jnp.float32),
  )

  for W, b in ((W1, b1), (W2, b2)):
    parts = edge_kernel(g, row, col)
    g = layer_call(parts, g, dinv, W, b.reshape(1, D))

  parts = edge_kernel(g, row, col)
  out = pl.pallas_call(
      functools.partial(_final_body, N, BLK),
      grid=(GRID,),
      in_specs=[
          pl.BlockSpec((NC, BLK, D), lambda i: (0, i, 0)),
          pl.BlockSpec((BLK, D), lambda i: (i, 0)),
          pl.BlockSpec((BLK, 1), lambda i: (i, 0)),
          pl.BlockSpec((D, D), lambda i: (0, 0)),
          pl.BlockSpec((1, D), lambda i: (0, 0)),
          pl.BlockSpec((D, OUT), lambda i: (0, 0)),
          pl.BlockSpec((1, OUT), lambda i: (0, 0)),
      ],
      out_specs=pl.BlockSpec((1, OUT), lambda i: (0, 0)),
      out_shape=jax.ShapeDtypeStruct((1, OUT), jnp.float32),
      scratch_shapes=[pltpu.VMEM((1, D), jnp.float32)],
  )(parts, g, dinv, W3, b3.reshape(1, D), Wout, bout.reshape(1, OUT))
  return out


# speedup vs baseline: 1.3062x; 1.0006x over previous
import functools

import jax
import jax.numpy as jnp
from jax import lax
from jax.experimental import pallas as pl
from jax.experimental.pallas import tpu as pltpu
from jax.experimental.pallas import tpu_sc as plsc

NC = 2
NS = 16
NW = NC * NS
CHUNK = 125


def _sc_mesh():
  return plsc.VectorSubcoreMesh(
      core_axis_name="c", subcore_axis_name="s", num_cores=NC, num_subcores=NS)


def _make_deg_kernel(E, NP):
  per_w = E // NW
  n_it = per_w // CHUNK
  rows_per_tile = NP // NS

  @functools.partial(
      pl.kernel,
      out_type=jax.ShapeDtypeStruct((NC, NP, 16), jnp.float32),
      mesh=_sc_mesh(),
      scratch_types=[
          pltpu.VMEM_SHARED((NP, 16), jnp.float32),
          pltpu.VMEM((n_it, CHUNK), jnp.int32),
          pltpu.VMEM((CHUNK, 16), jnp.float32),
          pltpu.VMEM((128, 16), jnp.float32),
          pltpu.SemaphoreType.DMA,
      ],
  )
  def deg_kernel(col_hbm, out_hbm, acc, cbuf, ones_buf, zbuf, ss):
    c = lax.axis_index("c")
    s = lax.axis_index("s")
    wid = c * NS + s
    zeros16 = jnp.zeros((16,), jnp.float32)
    ones16 = jnp.ones((16,), jnp.float32)

    def fill_body(i, _):
      zbuf[i] = zeros16
      return 0
    lax.fori_loop(0, 128, fill_body, 0)

    def ones_body(i, _):
      ones_buf[i] = ones16
      return 0
    lax.fori_loop(0, CHUNK, ones_body, 0)

    for k in range(rows_per_tile // 128):
      pltpu.sync_copy(zbuf, acc.at[pl.ds(s * rows_per_tile + k * 128, 128)])

    pltpu.sync_copy(col_hbm.at[wid], cbuf)
    plsc.subcore_barrier()

    def step(i, _):
      pltpu.sync_copy(ones_buf, acc.at[cbuf.at[i]], add=True)
      return 0
    lax.fori_loop(0, n_it, step, 0)

    plsc.subcore_barrier()
    pltpu.sync_copy(acc.at[pl.ds(s * rows_per_tile, rows_per_tile)],
                    out_hbm.at[c, pl.ds(s * rows_per_tile, rows_per_tile)])

  return deg_kernel


def _make_edge_kernel(E, NP, D):
  per_w = E // NW
  n_it = per_w // CHUNK
  rows_per_tile = NP // NS

  @functools.partial(
      pl.kernel,
      out_type=jax.ShapeDtypeStruct((NC, NP, D), jnp.float32),
      mesh=_sc_mesh(),
      scratch_types=[
          pltpu.VMEM_SHARED((NP, D), jnp.float32),
          pltpu.VMEM((1, CHUNK), jnp.int32),
          pltpu.VMEM((1, CHUNK), jnp.int32),
          pltpu.VMEM((1, CHUNK), jnp.int32),
          pltpu.VMEM((1, CHUNK), jnp.int32),
          pltpu.VMEM((CHUNK, D), jnp.float32),
          pltpu.VMEM((CHUNK, D), jnp.float32),
          pltpu.SemaphoreType.DMA,
          pltpu.SemaphoreType.DMA,
          pltpu.SemaphoreType.DMA,
          pltpu.SemaphoreType.DMA,
          pltpu.SemaphoreType.DMA,
          pltpu.SemaphoreType.DMA,
          pltpu.SemaphoreType.DMA,
          pltpu.SemaphoreType.DMA,
      ],
  )
  def edge_kernel(g_hbm, row_hbm, col_hbm, out_hbm, acc, rb0, rb1, cb0, cb1,
                  rows0, rows1, ir0, ir1, ic0, ic1, gs0, gs1, ss0, ss1):
    c = lax.axis_index("c")
    s = lax.axis_index("s")
    wid = c * NS + s
    zeros16 = jnp.zeros((16,), jnp.float32)


    def iload(src_hbm, j, buf, sem):
      pltpu.async_copy(src_hbm.at[wid, j], buf, sem)

    def iwait(buf, sem):
      pltpu.make_async_copy(row_hbm.at[0, 0], buf, sem).wait()

    def gather(ibuf, buf, sem):
      pltpu.async_copy(g_hbm.at[ibuf.at[0]], buf, sem)

    def gwait(ibuf, buf, sem):
      pltpu.make_async_copy(g_hbm.at[ibuf.at[0]], buf, sem).wait()

    def scat(ibuf, buf, sem):
      pltpu.async_copy(buf, acc.at[ibuf.at[0]], sem, add=True)

    def swait(ibuf, buf, sem):
      pltpu.make_async_copy(buf, acc.at[ibuf.at[0]], sem).wait()

    n_pair = n_it // 2
    iload(row_hbm, 0, rb0, ir0)
    iload(col_hbm, 0, cb0, ic0)
    iload(row_hbm, 1, rb1, ir1)
    iload(col_hbm, 1, cb1, ic1)

    def zero_body(i, _):
      for j in range(D // 16):
        rows0[i, pl.ds(j * 16, 16)] = zeros16
      return 0
    lax.fori_loop(0, CHUNK, zero_body, 0)
    iwait(rb1, ir1)
    gather(rb1, rows1, gs1)
    nz, rem = rows_per_tile // CHUNK, rows_per_tile % CHUNK
    for k in range(nz):
      pltpu.sync_copy(rows0, acc.at[pl.ds(s * rows_per_tile + k * CHUNK, CHUNK)])
    if rem:
      pltpu.sync_copy(rows0.at[pl.ds(0, rem)],
                      acc.at[pl.ds(s * rows_per_tile + nz * CHUNK, rem)])
    iwait(rb0, ir0)
    gather(rb0, rows0, gs0)
    plsc.subcore_barrier()

    def step(i, _):
      gwait(rb0, rows0, gs0)

      @pl.when(i < n_pair - 1)
      def _():
        iload(row_hbm, 2 * i + 2, rb0, ir0)
      iwait(cb0, ic0)
      scat(cb0, rows0, ss0)
      swait(cb0, rows0, ss0)

      @pl.when(i < n_pair - 1)
      def _():
        iload(col_hbm, 2 * i + 2, cb0, ic0)
        iwait(rb0, ir0)
        gather(rb0, rows0, gs0)

      gwait(rb1, rows1, gs1)

      @pl.when(i < n_pair - 1)
      def _():
        iload(row_hbm, 2 * i + 3, rb1, ir1)
      iwait(cb1, ic1)
      scat(cb1, rows1, ss1)
      swait(cb1, rows1, ss1)

      @pl.when(i < n_pair - 1)
      def _():
        iload(col_hbm, 2 * i + 3, cb1, ic1)
        iwait(rb1, ir1)
        gather(rb1, rows1, gs1)
      return 0
    lax.fori_loop(0, n_pair, step, 0)

    plsc.subcore_barrier()
    pltpu.sync_copy(acc.at[pl.ds(s * rows_per_tile, rows_per_tile)],
                    out_hbm.at[c, pl.ds(s * rows_per_tile, rows_per_tile)])

  return edge_kernel


def _prescale_body(degp_ref, x_ref, dinv_ref, g_ref):
  dsum = degp_ref[0] + degp_ref[1]
  deg = jnp.sum(dsum, axis=1, keepdims=True) + 1.0
  dinv = lax.rsqrt(deg)
  dinv_ref[...] = dinv
  g_ref[...] = x_ref[...] * dinv


def _layer_body(parts_ref, g_ref, dinv_ref, w_ref, b_ref, gout_ref):
  g = g_ref[...]
  dinv = dinv_ref[...]
  agg = (parts_ref[0] + parts_ref[1] + g) * dinv
  h = jnp.maximum(
      lax.dot_general(agg, w_ref[...], (((1,), (0,)), ((), ())),
                      preferred_element_type=jnp.float32) + b_ref[...], 0.0)
  gout_ref[...] = h * dinv


def _final_body(N, BLK, parts_ref, g_ref, dinv_ref, w_ref, b_ref, wo_ref,
                bo_ref, out_ref, acc_ref):
  i = pl.program_id(0)

  @pl.when(i == 0)
  def _():
    acc_ref[...] = jnp.zeros_like(acc_ref)

  g = g_ref[...]
  dinv = dinv_ref[...]
  agg = (parts_ref[0] + parts_ref[1] + g) * dinv
  h = jnp.maximum(
      lax.dot_general(agg, w_ref[...], (((1,), (0,)), ((), ())),
                      preferred_element_type=jnp.float32) + b_ref[...], 0.0)
  rid = lax.broadcasted_iota(jnp.int32, h.shape, 0) + i * BLK
  h = jnp.where(rid < N, h, 0.0)
  acc_ref[...] += jnp.sum(h, axis=0, keepdims=True)
  pooled = acc_ref[...] * (1.0 / N)
  out_ref[...] = lax.dot_general(pooled, wo_ref[...], (((1,), (0,)), ((), ())),
                                 preferred_element_type=jnp.float32) + bo_ref[...]


def kernel(x, edge_index, W1, b1, W2, b2, W3, b3, Wout, bout):
  N, D = x.shape
  E = edge_index.shape[1]
  OUT = Wout.shape[1]
  NP = ((N + 1023) // 1024) * 1024
  BLK = 1024
  GRID = NP // BLK
  per_w = E // NW
  n_it = per_w // CHUNK

  row = edge_index[0].reshape(NW, n_it, 1, CHUNK)
  col = edge_index[1].reshape(NW, n_it, 1, CHUNK)
  x_pad = jnp.zeros((NP, D), x.dtype).at[:N].set(x)

  deg_parts = _make_deg_kernel(E, NP)(edge_index[1].reshape(NW, -1, CHUNK))

  dinv, g = pl.pallas_call(
      _prescale_body,
      grid=(GRID,),
      in_specs=[
          pl.BlockSpec((NC, BLK, 16), lambda i: (0, i, 0)),
          pl.BlockSpec((BLK, D), lambda i: (i, 0)),
      ],
      out_specs=[
          pl.BlockSpec((BLK, 1), lambda i: (i, 0)),
          pl.BlockSpec((BLK, D), lambda i: (i, 0)),
      ],
      out_shape=[
          jax.ShapeDtypeStruct((NP, 1), jnp.float32),
          jax.ShapeDtypeStruct((NP, D), jnp.float32),
      ],
  )(deg_parts, x_pad)

  edge_kernel = _make_edge_kernel(E, NP, D)

  layer_call = pl.pallas_call(
      _layer_body,
      grid=(GRID,),
      in_specs=[
          pl.BlockSpec((NC, BLK, D), lambda i: (0, i, 0)),
          pl.BlockSpec((BLK, D), lambda i: (i, 0)),
          pl.BlockSpec((BLK, 1), lambda i: (i, 0)),
          pl.BlockSpec((D, D), lambda i: (0, 0)),
          pl.BlockSpec((1, D), lambda i: (0, 0)),
      ],
      out_specs=pl.BlockSpec((BLK, D), lambda i: (i, 0)),
      out_shape=jax.ShapeDtypeStruct((NP, D), jnp.float32),
  )

  for W, b in ((W1, b1), (W2, b2)):
    parts = edge_kernel(g, row, col)
    g = layer_call(parts, g, dinv, W, b.reshape(1, D))

  parts = edge_kernel(g, row, col)
  out = pl.pallas_call(
      functools.partial(_final_body, N, BLK),
      grid=(GRID,),
      in_specs=[
          pl.BlockSpec((NC, BLK, D), lambda i: (0, i, 0)),
          pl.BlockSpec((BLK, D), lambda i: (i, 0)),
          pl.BlockSpec((BLK, 1), lambda i: (i, 0)),
          pl.BlockSpec((D, D), lambda i: (0, 0)),
          pl.BlockSpec((1, D), lambda i: (0, 0)),
          pl.BlockSpec((D, OUT), lambda i: (0, 0)),
          pl.BlockSpec((1, OUT), lambda i: (0, 0)),
      ],
      out_specs=pl.BlockSpec((1, OUT), lambda i: (0, 0)),
      out_shape=jax.ShapeDtypeStruct((1, OUT), jnp.float32),
      scratch_shapes=[pltpu.VMEM((1, D), jnp.float32)],
  )(parts, g, dinv, W3, b3.reshape(1, D), Wout, bout.reshape(1, OUT))
  return out
